# baseline (speedup 1.0000x reference)
import functools

import jax
import jax.numpy as jnp
from jax import lax
from jax.experimental import pallas as pl
from jax.experimental.pallas import tpu as pltpu
from jax.experimental.pallas import tpu_sc as plsc

NUM_USERS = 1000000
NUM_ITEMS = 100000
EMB = 64
TEXT_DIM = 384
HID = 128
B = 4096
NNEG = 50
LANES = 128

_NC = 2
_NS = 16
_NW = _NC * _NS
_CH = 128
_JP = 56
_ITEM_ROWS = B * _JP
_HALF = 2
_HROWS = _ITEM_ROWS // _HALF
_CPW = _HROWS // (_NW * _CH)


_K1_ROWS = 2048


def _item_tower_body(text_ref, w1_ref, b1_ref, w2_ref, b2_ref, idt_ref, out_ref):
    h = jnp.dot(text_ref[...], w1_ref[...], preferred_element_type=jnp.float32)
    h = jnp.maximum(h + b1_ref[...], 0.0)
    p = jnp.dot(h, w2_ref[...], preferred_element_type=jnp.float32)
    v = p + b2_ref[...] + idt_ref[...].T
    out_ref[...] = jnp.concatenate([v, jnp.zeros_like(v)], axis=1)


def _item_tower(text_emb, W1, b1, W2, b2, item_id_emb):
    grid = pl.cdiv(NUM_ITEMS, _K1_ROWS)
    return pl.pallas_call(
        _item_tower_body,
        grid=(grid,),
        in_specs=[
            pl.BlockSpec((_K1_ROWS, TEXT_DIM), lambda i: (i, 0)),
            pl.BlockSpec((TEXT_DIM, HID), lambda i: (0, 0)),
            pl.BlockSpec((1, HID), lambda i: (0, 0)),
            pl.BlockSpec((HID, EMB), lambda i: (0, 0)),
            pl.BlockSpec((1, EMB), lambda i: (0, 0)),
            pl.BlockSpec((EMB, _K1_ROWS), lambda i: (0, i)),
        ],
        out_specs=pl.BlockSpec((_K1_ROWS, LANES), lambda i: (i, 0)),
        out_shape=jax.ShapeDtypeStruct((NUM_ITEMS, LANES), jnp.float32),
    )(text_emb, W1, b1.reshape(1, HID), W2, b2.reshape(1, EMB),
      item_id_emb.T)


_NB = 4


def _sc_gather_body(proj_hbm, idx_hbm, item_out, idx_v, b0, b1, b2, b3, *sems):
    gsem = sems[:_NB]
    wsem = sems[_NB:]
    bufs = (b0, b1, b2, b3)
    w = lax.axis_index("s") * _NC + lax.axis_index("c")
    ipw = _CPW * _CH
    pltpu.sync_copy(idx_hbm.at[pl.ds(pl.multiple_of(w * ipw, _CH), ipw)], idx_v)

    def gstart(c, k):
        sl = pl.ds(pl.multiple_of(c * _CH, _CH), _CH)
        pltpu.async_copy(proj_hbm.at[idx_v.at[sl]], bufs[k], gsem[k])

    def gwait(k):
        pltpu.make_async_copy(proj_hbm.at[idx_v.at[pl.ds(0, _CH)]], bufs[k],
                              gsem[k]).wait()

    def wstart(c, k):
        out_sl = pl.ds(pl.multiple_of((w * _CPW + c) * _CH, _CH), _CH)
        pltpu.async_copy(bufs[k], item_out.at[out_sl], wsem[k])

    def wwait(k):
        pltpu.make_async_copy(bufs[k], item_out.at[pl.ds(0, _CH)],
                              wsem[k]).wait()

    for k in range(_NB):
        gstart(k, k)

    def body(p, carry):
        c = _NB * p
        for k in range(_NB):
            gwait(k)
            wstart(c + k, k)
        for k in range(_NB):
            wwait(k)
            gstart(c + _NB + k, k)
        return carry

    lax.fori_loop(0, _CPW // _NB - 1, body, 0)
    for k in range(_NB):
        gwait(k)
        wstart(_CPW - _NB + k, k)
    for k in range(_NB):
        wwait(k)


def _sc_gather(proj, idx_all):
    mesh = plsc.VectorSubcoreMesh(core_axis_name="c", subcore_axis_name="s")
    kern = functools.partial(
        pl.kernel,
        mesh=mesh,
        out_type=jax.ShapeDtypeStruct((_HROWS, LANES), jnp.float32),
        scratch_types=(
            [pltpu.VMEM((_CPW * _CH,), jnp.int32)]
            + [pltpu.VMEM((_CH, LANES), jnp.float32) for _ in range(_NB)]
            + [pltpu.SemaphoreType.DMA for _ in range(2 * _NB)]
        ),
    )(_sc_gather_body)
    return kern(proj, idx_all)


_SB = 512


def _score_body(item_ref, user_ref, out_ref):
    out_ref[...] = jnp.sum(item_ref[...] * user_ref[...][:, None, :], axis=2)


def _score(item_g3, user_pad):
    return pl.pallas_call(
        _score_body,
        grid=(B // _HALF // _SB,),
        in_specs=[
            pl.BlockSpec((_SB, _JP, LANES), lambda i: (i, 0, 0)),
            pl.BlockSpec((_SB, LANES), lambda i: (i, 0)),
        ],
        out_specs=pl.BlockSpec((_SB, _JP), lambda i: (i, 0)),
        out_shape=jax.ShapeDtypeStruct((B // _HALF, _JP), jnp.float32),
    )(item_g3, user_pad)


def kernel(user_idx, pos_item_idx, neg_item_indices, user_emb, text_emb,
           W1, b1, W2, b2, item_id_emb):
    proj = _item_tower(text_emb, W1, b1, W2, b2, item_id_emb)
    idx_all = jnp.concatenate(
        [pos_item_idx[:, None], neg_item_indices,
         jnp.broadcast_to(pos_item_idx[:, None], (B, _JP - NNEG - 1))],
        axis=1).reshape(-1)
    uidx_grid = jnp.broadcast_to(user_idx[None, :], (EMB, B))
    user_vec = jnp.take_along_axis(user_emb.T, uidx_grid, axis=1).T
    user_pad = jnp.pad(user_vec, ((0, 0), (0, LANES - EMB)))
    bh = B // _HALF
    halves = []
    for h in range(_HALF):
        item_g = _sc_gather(proj, idx_all[h * _HROWS:(h + 1) * _HROWS])
        halves.append(_score(item_g.reshape(bh, _JP, LANES),
                             user_pad[h * bh:(h + 1) * bh]))
    scores = jnp.concatenate(halves, axis=0)
    return (scores[:, 0], scores[:, 1:NNEG + 1])


# speedup vs baseline: 1.0102x; 1.0102x over previous
import functools

import jax
import jax.numpy as jnp
from jax import lax
from jax.experimental import pallas as pl
from jax.experimental.pallas import tpu as pltpu
from jax.experimental.pallas import tpu_sc as plsc

NUM_USERS = 1000000
NUM_ITEMS = 100000
EMB = 64
TEXT_DIM = 384
HID = 128
B = 4096
NNEG = 50
LANES = 128

_NC = 2
_NS = 16
_NW = _NC * _NS
_CH = 128
_JP = 56
_ITEM_ROWS = B * _JP
_CPW = _ITEM_ROWS // (_NW * _CH)


_K1_ROWS = 2048


def _item_tower_body(text_ref, w1_ref, b1_ref, w2_ref, b2_ref, idt_ref, out_ref):
    h = jnp.dot(text_ref[...], w1_ref[...], preferred_element_type=jnp.float32)
    h = jnp.maximum(h + b1_ref[...], 0.0)
    p = jnp.dot(h, w2_ref[...], preferred_element_type=jnp.float32)
    v = p + b2_ref[...] + idt_ref[...].T
    out_ref[...] = jnp.concatenate([v, jnp.zeros_like(v)], axis=1)


def _item_tower(text_emb, W1, b1, W2, b2, item_id_emb):
    grid = pl.cdiv(NUM_ITEMS, _K1_ROWS)
    return pl.pallas_call(
        _item_tower_body,
        grid=(grid,),
        in_specs=[
            pl.BlockSpec((_K1_ROWS, TEXT_DIM), lambda i: (i, 0)),
            pl.BlockSpec((TEXT_DIM, HID), lambda i: (0, 0)),
            pl.BlockSpec((1, HID), lambda i: (0, 0)),
            pl.BlockSpec((HID, EMB), lambda i: (0, 0)),
            pl.BlockSpec((1, EMB), lambda i: (0, 0)),
            pl.BlockSpec((EMB, _K1_ROWS), lambda i: (0, i)),
        ],
        out_specs=pl.BlockSpec((_K1_ROWS, LANES), lambda i: (i, 0)),
        out_shape=jax.ShapeDtypeStruct((NUM_ITEMS, LANES), jnp.float32),
    )(text_emb, W1, b1.reshape(1, HID), W2, b2.reshape(1, EMB),
      item_id_emb.T)


_NB = 7


def _sc_gather_body(proj_hbm, idx_hbm, item_out, idx_v, *rest):
    bufs = rest[:_NB]
    gsem = rest[_NB:2 * _NB]
    wsem = rest[2 * _NB:]
    w = lax.axis_index("s") * _NC + lax.axis_index("c")
    ipw = _CPW * _CH
    pltpu.sync_copy(idx_hbm.at[pl.ds(pl.multiple_of(w * ipw, _CH), ipw)], idx_v)

    def gstart(c, k):
        sl = pl.ds(pl.multiple_of(c * _CH, _CH), _CH)
        pltpu.async_copy(proj_hbm.at[idx_v.at[sl]], bufs[k], gsem[k])

    def gwait(k):
        pltpu.make_async_copy(proj_hbm.at[idx_v.at[pl.ds(0, _CH)]], bufs[k],
                              gsem[k]).wait()

    def wstart(c, k):
        out_sl = pl.ds(pl.multiple_of((w * _CPW + c) * _CH, _CH), _CH)
        pltpu.async_copy(bufs[k], item_out.at[out_sl], wsem[k])

    def wwait(k):
        pltpu.make_async_copy(bufs[k], item_out.at[pl.ds(0, _CH)],
                              wsem[k]).wait()

    for k in range(_NB):
        gstart(k, k)

    def body(p, carry):
        c = _NB * p
        for k in range(_NB):
            gwait(k)
            wstart(c + k, k)
        for k in range(_NB):
            wwait(k)
            gstart(c + _NB + k, k)
        return carry

    lax.fori_loop(0, _CPW // _NB - 1, body, 0)
    for k in range(_NB):
        gwait(k)
        wstart(_CPW - _NB + k, k)
    for k in range(_NB):
        wwait(k)


def _sc_gather(proj, idx_all):
    mesh = plsc.VectorSubcoreMesh(core_axis_name="c", subcore_axis_name="s")
    kern = functools.partial(
        pl.kernel,
        mesh=mesh,
        out_type=jax.ShapeDtypeStruct((_ITEM_ROWS, LANES), jnp.float32),
        scratch_types=(
            [pltpu.VMEM((_CPW * _CH,), jnp.int32)]
            + [pltpu.VMEM((_CH, LANES), jnp.float32) for _ in range(_NB)]
            + [pltpu.SemaphoreType.DMA for _ in range(2 * _NB)]
        ),
    )(_sc_gather_body)
    return kern(proj, idx_all)


_SB = 512


def _score_body(item_ref, user_ref, out_ref):
    out_ref[...] = jnp.sum(item_ref[...] * user_ref[...][:, None, :], axis=2)


def _score(item_g3, user_pad):
    return pl.pallas_call(
        _score_body,
        grid=(B // _SB,),
        in_specs=[
            pl.BlockSpec((_SB, _JP, LANES), lambda i: (i, 0, 0)),
            pl.BlockSpec((_SB, LANES), lambda i: (i, 0)),
        ],
        out_specs=pl.BlockSpec((_SB, _JP), lambda i: (i, 0)),
        out_shape=jax.ShapeDtypeStruct((B, _JP), jnp.float32),
    )(item_g3, user_pad)


def kernel(user_idx, pos_item_idx, neg_item_indices, user_emb, text_emb,
           W1, b1, W2, b2, item_id_emb):
    proj = _item_tower(text_emb, W1, b1, W2, b2, item_id_emb)
    idx_all = jnp.concatenate(
        [pos_item_idx[:, None], neg_item_indices,
         jnp.broadcast_to(pos_item_idx[:, None], (B, _JP - NNEG - 1))],
        axis=1).reshape(-1)
    item_g = _sc_gather(proj, idx_all)
    uidx_grid = jnp.broadcast_to(user_idx[None, :], (EMB, B))
    user_vec = jnp.take_along_axis(user_emb.T, uidx_grid, axis=1).T
    user_pad = jnp.pad(user_vec, ((0, 0), (0, LANES - EMB)))
    scores = _score(item_g.reshape(B, _JP, LANES), user_pad)
    return (scores[:, 0], scores[:, 1:NNEG + 1])
